# Initial kernel scaffold; baseline (speedup 1.0000x reference)
import jax
import jax.numpy as jnp
from jax.experimental import pallas as pl


def kernel(x, dutyCycle):
    raise NotImplementedError("write your pallas kernel here")


# speedup vs baseline: 24.3439x; 24.3439x over previous
import functools

import jax
import jax.numpy as jnp
from jax import lax
from jax.experimental import pallas as pl
from jax.experimental.pallas import tpu as pltpu
from jax.experimental.pallas import tpu_sc as plsc

B = 16
C = 96
HW = 112 * 112
NV = HW // 16
KWIN = 120422
TD = float(KWIN) / (C * HW)

NB = 2048
LO1 = -16.0
W1 = 32.0 / NB
S1 = 1.0 / W1
NF = NB - 2
S2 = NF / W1
W2 = W1 / NF

CH_PER_W = C // 2


def _body(x_hbm, a1_hbm, a2_hbm, ibf_hbm, out_hbm, data, hist2d, hist1d, nbh,
          a1t, a2t, ibft, pbuf, sbuf, rbuf, lbuf, shist):
    cidx = lax.axis_index("c")
    sidx = lax.axis_index("s")
    b = cidx * 8 + sidx // 2
    h = sidx % 2
    c0 = h * CH_PER_W
    prow = sidx - h
    poff = h * 16

    def put_params(vec_f32):
        sbuf[...] = plsc.bitcast(vec_f32, jnp.int32)
        pltpu.sync_copy(sbuf, shist.at[sidx, pl.ds(0, 16)])
        pltpu.sync_copy(sbuf, shist.at[sidx, pl.ds(16, 16)])

    def get_params():
        pltpu.sync_copy(shist.at[prow, pl.ds(poff, 16)], pbuf)
        return plsc.bitcast(pbuf[...], jnp.float32)

    lanes = lax.iota(jnp.int32, 16)
    ones16 = jnp.ones((16,), jnp.int32)

    pltpu.sync_copy(a1_hbm, a1t)
    pltpu.sync_copy(a2_hbm, a2t)
    pltpu.sync_copy(ibf_hbm, ibft)

    def zero_hist(_i, _):
        hist2d[pl.ds(_i * 16, 16)] = jnp.zeros((16,), jnp.int32)
        return 0

    def hist_pass(at_ref, off_vec):
        def chan_body(ci, _):
            ch = c0 + ci
            pltpu.sync_copy(x_hbm.at[b, ch], data)
            a = at_ref[ch]

            def vb(i, _):
                v = data[pl.ds(i * 16, 16)]
                t = v * a + off_vec
                t = jnp.minimum(jnp.maximum(t, 0.0), float(NB - 1))
                idx = t.astype(jnp.int32) * 16 + lanes
                plsc.addupdate_scatter(hist2d, (idx,), ones16)
                return 0
            lax.fori_loop(0, NV, vb, 0)
            return 0
        lax.fori_loop(0, CH_PER_W, chan_body, 0)

    def reduce_lanes(blk, _):
        row0 = (blk * 16 + lanes) * 16

        def lb(l, acc):
            return acc + plsc.load_gather(hist2d, (row0 + l,))
        acc = lax.fori_loop(0, 16, lb, jnp.zeros((16,), jnp.int32))
        hist1d[pl.ds(blk * 16, 16)] = acc
        return 0

    def merge_pair(blk, _):
        sl = pl.ds(blk * 16, 16)
        hist1d[sl] = hist1d[sl] + nbh[sl]
        return 0

    def scan_hist(target):
        def sb(j, carry):
            run, cnt, mx = carry
            blk = NB // 16 - 1 - j
            v = hist1d[pl.ds(blk * 16, 16)]
            cs = plsc.cumsum(lax.rev(v, (0,))) + jnp.broadcast_to(run, (16,))
            ge = cs >= target
            cnt = cnt + jnp.sum(ge.astype(jnp.int32))
            mx = jnp.maximum(mx, jnp.max(jnp.where(ge, 0, cs)))
            return (cs[15], cnt, mx)
        _, cnt, mx = lax.fori_loop(
            0, NB // 16, sb, (jnp.int32(0), jnp.int32(0), jnp.int32(0)))
        return cnt - 1, mx

    lax.fori_loop(0, NB, zero_hist, 0)
    off1 = jnp.full((16,), -LO1 * S1, jnp.float32)
    hist_pass(a1t, off1)
    lax.fori_loop(0, NB // 16, reduce_lanes, 0)

    @pl.when(h == 1)
    def _():
        pltpu.sync_copy(hist1d, shist.at[sidx])
    plsc.subcore_barrier()

    @pl.when(h == 0)
    def _():
        pltpu.sync_copy(shist.at[sidx + 1], nbh)
        lax.fori_loop(0, NB // 16, merge_pair, 0)
        jstar, cabove = scan_hist(jnp.int32(KWIN))
        rem = jnp.int32(KWIN) - cabove
        rbuf[...] = jnp.broadcast_to(rem, (16,))
        lo2 = jnp.float32(LO1) + jstar.astype(jnp.float32) * jnp.float32(W1)
        lbuf[...] = jnp.broadcast_to(lo2, (16,))
        off2 = 1.0 - lo2 * jnp.float32(S2)
        put_params(jnp.broadcast_to(off2, (16,)))
    plsc.subcore_barrier()

    off2v = get_params()
    lax.fori_loop(0, NB, zero_hist, 0)
    hist_pass(a2t, off2v)
    lax.fori_loop(0, NB // 16, reduce_lanes, 0)

    @pl.when(h == 1)
    def _():
        pltpu.sync_copy(hist1d, shist.at[sidx])
    plsc.subcore_barrier()

    @pl.when(h == 0)
    def _():
        pltpu.sync_copy(shist.at[sidx + 1], nbh)
        lax.fori_loop(0, NB // 16, merge_pair, 0)
        blk0 = hist1d[pl.ds(0, 16)]
        hist1d[pl.ds(0, 16)] = jnp.where(lanes == 0, 0, blk0)
        blkt = hist1d[pl.ds(NB - 16, 16)]
        hist1d[pl.ds(NB - 16, 16)] = jnp.where(lanes == 15, 0, blkt)
        rem = rbuf[...][0]
        mstar, _ = scan_hist(rem)
        lo2 = lbuf[...][0]
        tau = lo2 + (mstar - 1).astype(jnp.float32) * jnp.float32(W2)
        put_params(jnp.broadcast_to(tau, (16,)))
    plsc.subcore_barrier()

    tauv = get_params()

    def chan3(ci, _):
        ch = c0 + ci
        pltpu.sync_copy(x_hbm.at[b, ch], data)
        th = tauv * ibft[ch]

        def vb(i, _):
            sl = pl.ds(i * 16, 16)
            v = data[sl]
            data[sl] = jnp.where(v >= th, v, 0.0)
            return 0
        lax.fori_loop(0, NV, vb, 0)
        pltpu.sync_copy(data, out_hbm.at[b, ch])
        return 0
    lax.fori_loop(0, CH_PER_W, chan3, 0)


@functools.partial(
    pl.kernel,
    out_type=jax.ShapeDtypeStruct((B, C, HW), jnp.float32),
    mesh=plsc.VectorSubcoreMesh(core_axis_name="c", subcore_axis_name="s"),
    scratch_types=[
        pltpu.VMEM((HW,), jnp.float32),
        pltpu.VMEM((NB * 16,), jnp.int32),
        pltpu.VMEM((NB,), jnp.int32),
        pltpu.VMEM((NB,), jnp.int32),
        pltpu.VMEM((C, 16), jnp.float32),
        pltpu.VMEM((C, 16), jnp.float32),
        pltpu.VMEM((C, 16), jnp.float32),
        pltpu.VMEM((16,), jnp.int32),
        pltpu.VMEM((16,), jnp.int32),
        pltpu.VMEM((16,), jnp.int32),
        pltpu.VMEM((16,), jnp.float32),
        pltpu.VMEM_SHARED((16, NB), jnp.int32),
    ],
    compiler_params=pltpu.CompilerParams(needs_layout_passes=False),
)
def _kwinners_sc(x_hbm, a1_hbm, a2_hbm, ibf_hbm, out_hbm, *scratch):
    _body(x_hbm, a1_hbm, a2_hbm, ibf_hbm, out_hbm, *scratch)


def kernel(x, dutyCycle):
    x2 = x.reshape(B, C, HW)
    bf = jnp.exp((jnp.float32(TD) - dutyCycle.astype(jnp.float32))).reshape(C, 1)
    a1 = jnp.broadcast_to(bf * jnp.float32(S1), (C, 16))
    a2 = jnp.broadcast_to(bf * jnp.float32(S2), (C, 16))
    ibf = jnp.broadcast_to(1.0 / bf, (C, 16))
    out = _kwinners_sc(x2, a1, a2, ibf)
    return out.reshape(x.shape)


# speedup vs baseline: 30.4773x; 1.2519x over previous
import functools

import jax
import jax.numpy as jnp
from jax import lax
from jax.experimental import pallas as pl
from jax.experimental.pallas import tpu as pltpu
from jax.experimental.pallas import tpu_sc as plsc

B = 16
C = 96
HW = 112 * 112
NV = HW // 16
KWIN = 120422
TD = float(KWIN) / (C * HW)

NB = 2048
LO1 = -16.0
W1 = 32.0 / NB
S1 = 1.0 / W1
NF = NB - 2
S2 = NF / W1
W2 = W1 / NF

CH_PER_W = C // 2


def _body(x_hbm, a1_hbm, a2_hbm, ibf_hbm, out_hbm, data0, data1, hist2d,
          hist1d, nbh, a1t, a2t, ibft, pbuf, sbuf, rbuf, lbuf, shist,
          sin0, sin1, sout0, sout1):
    cidx = lax.axis_index("c")
    sidx = lax.axis_index("s")
    b = cidx * 8 + sidx // 2
    h = sidx % 2
    c0 = h * CH_PER_W
    prow = sidx - h
    poff = h * 16

    def put_params(vec_f32):
        sbuf[...] = plsc.bitcast(vec_f32, jnp.int32)
        pltpu.sync_copy(sbuf, shist.at[sidx, pl.ds(0, 16)])
        pltpu.sync_copy(sbuf, shist.at[sidx, pl.ds(16, 16)])

    def get_params():
        pltpu.sync_copy(shist.at[prow, pl.ds(poff, 16)], pbuf)
        return plsc.bitcast(pbuf[...], jnp.float32)

    lanes = lax.iota(jnp.int32, 16)
    ones16 = jnp.ones((16,), jnp.int32)

    pltpu.sync_copy(a1_hbm, a1t)
    pltpu.sync_copy(a2_hbm, a2t)
    pltpu.sync_copy(ibf_hbm, ibft)

    def zero_hist(_i, _):
        hist2d[pl.ds(_i * 16, 16)] = jnp.zeros((16,), jnp.int32)
        return 0

    bufs = (data0, data1)
    sins = (sin0, sin1)
    souts = (sout0, sout1)

    def hist_pass(at_ref, off_vec):
        def process(buf, g):
            a = at_ref[c0 + g]

            def vb(iv, _):
                base = iv * 128
                for u in range(8):
                    v = buf[pl.ds(base + u * 16, 16)]
                    t = v * a + off_vec
                    t = jnp.minimum(jnp.maximum(t, 0.0), float(NB - 1))
                    idx = t.astype(jnp.int32) * 16 + lanes
                    plsc.addupdate_scatter(hist2d, (idx,), ones16)
                return 0
            lax.fori_loop(0, NV // 8, vb, 0)

        pltpu.async_copy(x_hbm.at[b, c0], data0, sin0)

        def outer(t, _):
            for q in range(2):
                g = t * 2 + q

                @pl.when(g + 1 < CH_PER_W)
                def _():
                    pltpu.async_copy(x_hbm.at[b, c0 + g + 1],
                                     bufs[1 - q], sins[1 - q])
                pltpu.make_async_copy(x_hbm.at[b, c0 + g],
                                     bufs[q], sins[q]).wait()
                process(bufs[q], g)
            return 0
        lax.fori_loop(0, CH_PER_W // 2, outer, 0)

    def reduce_lanes(blk, _):
        row0 = (blk * 16 + lanes) * 16

        def lb(l, acc):
            return acc + plsc.load_gather(hist2d, (row0 + l,))
        acc = lax.fori_loop(0, 16, lb, jnp.zeros((16,), jnp.int32))
        hist1d[pl.ds(blk * 16, 16)] = acc
        return 0

    def merge_pair(blk, _):
        sl = pl.ds(blk * 16, 16)
        hist1d[sl] = hist1d[sl] + nbh[sl]
        return 0

    def scan_hist(target):
        def sb(j, carry):
            run, cnt, mx = carry
            blk = NB // 16 - 1 - j
            v = hist1d[pl.ds(blk * 16, 16)]
            cs = plsc.cumsum(lax.rev(v, (0,))) + jnp.broadcast_to(run, (16,))
            ge = cs >= target
            cnt = cnt + jnp.sum(ge.astype(jnp.int32))
            mx = jnp.maximum(mx, jnp.max(jnp.where(ge, 0, cs)))
            return (cs[15], cnt, mx)
        _, cnt, mx = lax.fori_loop(
            0, NB // 16, sb, (jnp.int32(0), jnp.int32(0), jnp.int32(0)))
        return cnt - 1, mx

    lax.fori_loop(0, NB, zero_hist, 0)
    off1 = jnp.full((16,), -LO1 * S1, jnp.float32)
    hist_pass(a1t, off1)
    lax.fori_loop(0, NB // 16, reduce_lanes, 0)

    @pl.when(h == 1)
    def _():
        pltpu.sync_copy(hist1d, shist.at[sidx])
    plsc.subcore_barrier()

    @pl.when(h == 0)
    def _():
        pltpu.sync_copy(shist.at[sidx + 1], nbh)
        lax.fori_loop(0, NB // 16, merge_pair, 0)
        jstar, cabove = scan_hist(jnp.int32(KWIN))
        rem = jnp.int32(KWIN) - cabove
        rbuf[...] = jnp.broadcast_to(rem, (16,))
        lo2 = jnp.float32(LO1) + jstar.astype(jnp.float32) * jnp.float32(W1)
        lbuf[...] = jnp.broadcast_to(lo2, (16,))
        off2 = 1.0 - lo2 * jnp.float32(S2)
        put_params(jnp.broadcast_to(off2, (16,)))
    plsc.subcore_barrier()

    off2v = get_params()
    lax.fori_loop(0, NB, zero_hist, 0)
    hist_pass(a2t, off2v)
    lax.fori_loop(0, NB // 16, reduce_lanes, 0)

    @pl.when(h == 1)
    def _():
        pltpu.sync_copy(hist1d, shist.at[sidx])
    plsc.subcore_barrier()

    @pl.when(h == 0)
    def _():
        pltpu.sync_copy(shist.at[sidx + 1], nbh)
        lax.fori_loop(0, NB // 16, merge_pair, 0)
        blk0 = hist1d[pl.ds(0, 16)]
        hist1d[pl.ds(0, 16)] = jnp.where(lanes == 0, 0, blk0)
        blkt = hist1d[pl.ds(NB - 16, 16)]
        hist1d[pl.ds(NB - 16, 16)] = jnp.where(lanes == 15, 0, blkt)
        rem = rbuf[...][0]
        mstar, _ = scan_hist(rem)
        lo2 = lbuf[...][0]
        tau = lo2 + (mstar - 1).astype(jnp.float32) * jnp.float32(W2)
        put_params(jnp.broadcast_to(tau, (16,)))
    plsc.subcore_barrier()

    tauv = get_params()

    pltpu.async_copy(x_hbm.at[b, c0], data0, sin0)

    def outer3(t, _):
        for q in range(2):
            g = t * 2 + q

            @pl.when(g >= 1)
            def _():
                pltpu.make_async_copy(bufs[1 - q],
                                      out_hbm.at[b, c0 + g - 1],
                                      souts[1 - q]).wait()

            @pl.when(g + 1 < CH_PER_W)
            def _():
                pltpu.async_copy(x_hbm.at[b, c0 + g + 1],
                                 bufs[1 - q], sins[1 - q])
            pltpu.make_async_copy(x_hbm.at[b, c0 + g],
                                  bufs[q], sins[q]).wait()
            buf = bufs[q]
            th = tauv * ibft[c0 + g]

            def vb(iv, _):
                base = iv * 128
                for u in range(8):
                    sl = pl.ds(base + u * 16, 16)
                    v = buf[sl]
                    buf[sl] = jnp.where(v >= th, v, 0.0)
                return 0
            lax.fori_loop(0, NV // 8, vb, 0)
            pltpu.async_copy(buf, out_hbm.at[b, c0 + g], souts[q])
        return 0
    lax.fori_loop(0, CH_PER_W // 2, outer3, 0)
    pltpu.make_async_copy(data1, out_hbm.at[b, c0 + CH_PER_W - 1],
                          sout1).wait()


@functools.partial(
    pl.kernel,
    out_type=jax.ShapeDtypeStruct((B, C, HW), jnp.float32),
    mesh=plsc.VectorSubcoreMesh(core_axis_name="c", subcore_axis_name="s"),
    scratch_types=[
        pltpu.VMEM((HW,), jnp.float32),
        pltpu.VMEM((HW,), jnp.float32),
        pltpu.VMEM((NB * 16,), jnp.int32),
        pltpu.VMEM((NB,), jnp.int32),
        pltpu.VMEM((NB,), jnp.int32),
        pltpu.VMEM((C, 16), jnp.float32),
        pltpu.VMEM((C, 16), jnp.float32),
        pltpu.VMEM((C, 16), jnp.float32),
        pltpu.VMEM((16,), jnp.int32),
        pltpu.VMEM((16,), jnp.int32),
        pltpu.VMEM((16,), jnp.int32),
        pltpu.VMEM((16,), jnp.float32),
        pltpu.VMEM_SHARED((16, NB), jnp.int32),
        pltpu.SemaphoreType.DMA,
        pltpu.SemaphoreType.DMA,
        pltpu.SemaphoreType.DMA,
        pltpu.SemaphoreType.DMA,
    ],
    compiler_params=pltpu.CompilerParams(needs_layout_passes=False),
)
def _kwinners_sc(x_hbm, a1_hbm, a2_hbm, ibf_hbm, out_hbm, *scratch):
    _body(x_hbm, a1_hbm, a2_hbm, ibf_hbm, out_hbm, *scratch)


def kernel(x, dutyCycle):
    x2 = x.reshape(B, C, HW)
    bf = jnp.exp((jnp.float32(TD) - dutyCycle.astype(jnp.float32))).reshape(C, 1)
    a1 = jnp.broadcast_to(bf * jnp.float32(S1), (C, 16))
    a2 = jnp.broadcast_to(bf * jnp.float32(S2), (C, 16))
    ibf = jnp.broadcast_to(1.0 / bf, (C, 16))
    out = _kwinners_sc(x2, a1, a2, ibf)
    return out.reshape(x.shape)


# speedup vs baseline: 96.8530x; 3.1779x over previous
import functools

import jax
import jax.numpy as jnp
from jax import lax
from jax.experimental import pallas as pl
from jax.experimental.pallas import tpu as pltpu
from jax.experimental.pallas import tpu_sc as plsc

B = 16
C = 96
HW = 112 * 112
NV = HW // 16
KWIN = 120422
TD = float(KWIN) / (C * HW)

NB = 2048
LO1 = -16.0
W1 = 32.0 / NB
S1 = 1.0 / W1
NF = NB - 2
S2 = NF / W1
W2 = W1 / NF

CH_PER_W = C // 2


def _body(x_hbm, a1_hbm, a2_hbm, ibf_hbm, out_hbm, data0, data1, hist2d,
          hist1d, nbh, a1t, a2t, ibft, pbuf, sbuf, rbuf, lbuf, shist,
          sin0, sin1, sout0, sout1):
    cidx = lax.axis_index("c")
    sidx = lax.axis_index("s")
    b = cidx * 8 + sidx // 2
    h = sidx % 2
    c0 = h * CH_PER_W
    prow = sidx - h
    poff = h * 16

    def put_params(vec_f32):
        sbuf[...] = plsc.bitcast(vec_f32, jnp.int32)
        pltpu.sync_copy(sbuf, shist.at[sidx, pl.ds(0, 16)])
        pltpu.sync_copy(sbuf, shist.at[sidx, pl.ds(16, 16)])

    def get_params():
        pltpu.sync_copy(shist.at[prow, pl.ds(poff, 16)], pbuf)
        return plsc.bitcast(pbuf[...], jnp.float32)

    lanes = lax.iota(jnp.int32, 16)
    ones16 = jnp.ones((16,), jnp.int32)

    pltpu.sync_copy(a1_hbm, a1t)
    pltpu.sync_copy(a2_hbm, a2t)
    pltpu.sync_copy(ibf_hbm, ibft)

    def zero_hist():
        @plsc.parallel_loop(0, NB, step=1, unroll=8)
        def _zh(i):
            hist2d[pl.ds(i * 16, 16)] = jnp.zeros((16,), jnp.int32)

    bufs = (data0, data1)
    sins = (sin0, sin1)
    souts = (sout0, sout1)

    def hist_pass(at_ref, off_vec):
        def process(buf, g):
            a = at_ref[c0 + g]

            @plsc.parallel_loop(0, NV, step=1, unroll=8)
            def _vb(i):
                v = buf[pl.ds(i * 16, 16)]
                t = v * a + off_vec
                t = jnp.minimum(jnp.maximum(t, 0.0), float(NB - 1))
                idx = t.astype(jnp.int32) * 16 + lanes
                plsc.addupdate_scatter(hist2d, (idx,), ones16)

        pltpu.async_copy(x_hbm.at[b, c0], data0, sin0)

        def outer(t, _):
            for q in range(2):
                g = t * 2 + q

                @pl.when(g + 1 < CH_PER_W)
                def _():
                    pltpu.async_copy(x_hbm.at[b, c0 + g + 1],
                                     bufs[1 - q], sins[1 - q])
                pltpu.make_async_copy(x_hbm.at[b, c0 + g],
                                     bufs[q], sins[q]).wait()
                process(bufs[q], g)
            return 0
        lax.fori_loop(0, CH_PER_W // 2, outer, 0)

    def reduce_lanes():
        @plsc.parallel_loop(0, NB // 16, step=1, unroll=2)
        def _rb(blk):
            row0 = (blk * 16 + lanes) * 16
            g = [plsc.load_gather(hist2d, (row0 + l,)) for l in range(16)]
            while len(g) > 1:
                g = [g[i] + g[i + 1] for i in range(0, len(g), 2)]
            hist1d[pl.ds(blk * 16, 16)] = g[0]

    def merge_pair():
        @plsc.parallel_loop(0, NB // 16, step=1, unroll=4)
        def _mb(blk):
            sl = pl.ds(blk * 16, 16)
            hist1d[sl] = hist1d[sl] + nbh[sl]

    def scan_hist(target):
        def sb(j, carry):
            run, cnt, mx = carry
            blk = NB // 16 - 1 - j
            v = hist1d[pl.ds(blk * 16, 16)]
            cs = plsc.cumsum(lax.rev(v, (0,))) + jnp.broadcast_to(run, (16,))
            ge = cs >= target
            cnt = cnt + jnp.sum(ge.astype(jnp.int32))
            mx = jnp.maximum(mx, jnp.max(jnp.where(ge, 0, cs)))
            return (cs[15], cnt, mx)
        _, cnt, mx = lax.fori_loop(
            0, NB // 16, sb, (jnp.int32(0), jnp.int32(0), jnp.int32(0)))
        return cnt - 1, mx

    zero_hist()
    off1 = jnp.full((16,), -LO1 * S1, jnp.float32)
    hist_pass(a1t, off1)
    reduce_lanes()

    @pl.when(h == 1)
    def _():
        pltpu.sync_copy(hist1d, shist.at[sidx])
    plsc.subcore_barrier()

    @pl.when(h == 0)
    def _():
        pltpu.sync_copy(shist.at[sidx + 1], nbh)
        merge_pair()
        jstar, cabove = scan_hist(jnp.int32(KWIN))
        rem = jnp.int32(KWIN) - cabove
        rbuf[...] = jnp.broadcast_to(rem, (16,))
        lo2 = jnp.float32(LO1) + jstar.astype(jnp.float32) * jnp.float32(W1)
        lbuf[...] = jnp.broadcast_to(lo2, (16,))
        off2 = 1.0 - lo2 * jnp.float32(S2)
        put_params(jnp.broadcast_to(off2, (16,)))
    plsc.subcore_barrier()

    off2v = get_params()
    zero_hist()
    hist_pass(a2t, off2v)
    reduce_lanes()

    @pl.when(h == 1)
    def _():
        pltpu.sync_copy(hist1d, shist.at[sidx])
    plsc.subcore_barrier()

    @pl.when(h == 0)
    def _():
        pltpu.sync_copy(shist.at[sidx + 1], nbh)
        merge_pair()
        blk0 = hist1d[pl.ds(0, 16)]
        hist1d[pl.ds(0, 16)] = jnp.where(lanes == 0, 0, blk0)
        blkt = hist1d[pl.ds(NB - 16, 16)]
        hist1d[pl.ds(NB - 16, 16)] = jnp.where(lanes == 15, 0, blkt)
        rem = rbuf[...][0]
        mstar, _ = scan_hist(rem)
        lo2 = lbuf[...][0]
        tau = lo2 + (mstar - 1).astype(jnp.float32) * jnp.float32(W2)
        put_params(jnp.broadcast_to(tau, (16,)))
    plsc.subcore_barrier()

    tauv = get_params()

    pltpu.async_copy(x_hbm.at[b, c0], data0, sin0)

    def outer3(t, _):
        for q in range(2):
            g = t * 2 + q

            @pl.when(g >= 1)
            def _():
                pltpu.make_async_copy(bufs[1 - q],
                                      out_hbm.at[b, c0 + g - 1],
                                      souts[1 - q]).wait()

            @pl.when(g + 1 < CH_PER_W)
            def _():
                pltpu.async_copy(x_hbm.at[b, c0 + g + 1],
                                 bufs[1 - q], sins[1 - q])
            pltpu.make_async_copy(x_hbm.at[b, c0 + g],
                                  bufs[q], sins[q]).wait()
            buf = bufs[q]
            th = tauv * ibft[c0 + g]

            @plsc.parallel_loop(0, NV, step=1, unroll=8)
            def _vb(i):
                sl = pl.ds(i * 16, 16)
                v = buf[sl]
                buf[sl] = jnp.where(v >= th, v, 0.0)
            pltpu.async_copy(buf, out_hbm.at[b, c0 + g], souts[q])
        return 0
    lax.fori_loop(0, CH_PER_W // 2, outer3, 0)
    pltpu.make_async_copy(data1, out_hbm.at[b, c0 + CH_PER_W - 1],
                          sout1).wait()


@functools.partial(
    pl.kernel,
    out_type=jax.ShapeDtypeStruct((B, C, HW), jnp.float32),
    mesh=plsc.VectorSubcoreMesh(core_axis_name="c", subcore_axis_name="s"),
    scratch_types=[
        pltpu.VMEM((HW,), jnp.float32),
        pltpu.VMEM((HW,), jnp.float32),
        pltpu.VMEM((NB * 16,), jnp.int32),
        pltpu.VMEM((NB,), jnp.int32),
        pltpu.VMEM((NB,), jnp.int32),
        pltpu.VMEM((C, 16), jnp.float32),
        pltpu.VMEM((C, 16), jnp.float32),
        pltpu.VMEM((C, 16), jnp.float32),
        pltpu.VMEM((16,), jnp.int32),
        pltpu.VMEM((16,), jnp.int32),
        pltpu.VMEM((16,), jnp.int32),
        pltpu.VMEM((16,), jnp.float32),
        pltpu.VMEM_SHARED((16, NB), jnp.int32),
        pltpu.SemaphoreType.DMA,
        pltpu.SemaphoreType.DMA,
        pltpu.SemaphoreType.DMA,
        pltpu.SemaphoreType.DMA,
    ],
    compiler_params=pltpu.CompilerParams(needs_layout_passes=False),
)
def _kwinners_sc(x_hbm, a1_hbm, a2_hbm, ibf_hbm, out_hbm, *scratch):
    _body(x_hbm, a1_hbm, a2_hbm, ibf_hbm, out_hbm, *scratch)


def kernel(x, dutyCycle):
    x2 = x.reshape(B, C, HW)
    bf = jnp.exp((jnp.float32(TD) - dutyCycle.astype(jnp.float32))).reshape(C, 1)
    a1 = jnp.broadcast_to(bf * jnp.float32(S1), (C, 16))
    a2 = jnp.broadcast_to(bf * jnp.float32(S2), (C, 16))
    ibf = jnp.broadcast_to(1.0 / bf, (C, 16))
    out = _kwinners_sc(x2, a1, a2, ibf)
    return out.reshape(x.shape)


# speedup vs baseline: 147.8250x; 1.5263x over previous
import functools

import jax
import jax.numpy as jnp
from jax import lax
from jax.experimental import pallas as pl
from jax.experimental.pallas import tpu as pltpu
from jax.experimental.pallas import tpu_sc as plsc

B = 16
C = 96
HW = 112 * 112
NV = HW // 16
KWIN = 120422
TD = float(KWIN) / (C * HW)

NB = 2048
LO1 = -16.0
W1 = 32.0 / NB
S1 = 1.0 / W1
NF = NB - 2
S2 = NF / W1
W2 = W1 / NF

CH_PER_W = C // 2


def _body(x_hbm, a1_hbm, a2_hbm, ibf_hbm, out_hbm, data0, data1, hist2d,
          hist1d, nbh, a1t, a2t, ibft, pbuf, sbuf, rbuf, lbuf, shist,
          sin0, sin1, sout0, sout1):
    cidx = lax.axis_index("c")
    sidx = lax.axis_index("s")
    b = cidx * 8 + sidx // 2
    h = sidx % 2
    c0 = h * CH_PER_W
    prow = sidx - h
    poff = h * 16

    def put_params(vec_f32):
        sbuf[...] = plsc.bitcast(vec_f32, jnp.int32)
        pltpu.sync_copy(sbuf, shist.at[sidx, pl.ds(0, 16)])
        pltpu.sync_copy(sbuf, shist.at[sidx, pl.ds(16, 16)])

    def get_params():
        pltpu.sync_copy(shist.at[prow, pl.ds(poff, 16)], pbuf)
        return plsc.bitcast(pbuf[...], jnp.float32)

    lanes = lax.iota(jnp.int32, 16)
    ones16 = jnp.ones((16,), jnp.int32)

    pltpu.sync_copy(a1_hbm, a1t)
    pltpu.sync_copy(a2_hbm, a2t)
    pltpu.sync_copy(ibf_hbm, ibft)

    def zero_hist():
        @plsc.parallel_loop(0, NB, step=1, unroll=8)
        def _zh(i):
            hist2d[pl.ds(i * 16, 16)] = jnp.zeros((16,), jnp.int32)

    bufs = (data0, data1)
    sins = (sin0, sin1)
    souts = (sout0, sout1)

    def hist_pass(at_ref, off_vec):
        def process(buf, g):
            a = at_ref[c0 + g]

            @plsc.parallel_loop(0, 112, step=1, unroll=2)
            def _vb(r):
                for u in range(7):
                    v = buf[r, pl.ds(u * 16, 16)]
                    t = v * a + off_vec
                    t = jnp.minimum(jnp.maximum(t, 0.0), float(NB - 1))
                    idx = t.astype(jnp.int32) * 16 + lanes
                    plsc.addupdate_scatter(hist2d, (idx,), ones16)

        pltpu.async_copy(x_hbm.at[b * C + c0], data0, sin0)

        def outer(t, _):
            for q in range(2):
                g = t * 2 + q

                @pl.when(g + 1 < CH_PER_W)
                def _():
                    pltpu.async_copy(x_hbm.at[b * C + c0 + g + 1],
                                     bufs[1 - q], sins[1 - q])
                pltpu.make_async_copy(x_hbm.at[b * C + c0 + g],
                                     bufs[q], sins[q]).wait()
                process(bufs[q], g)
            return 0
        lax.fori_loop(0, CH_PER_W // 2, outer, 0)

    def reduce_lanes():
        @plsc.parallel_loop(0, NB // 16, step=1, unroll=2)
        def _rb(blk):
            row0 = (blk * 16 + lanes) * 16
            g = [plsc.load_gather(hist2d, (row0 + l,)) for l in range(16)]
            while len(g) > 1:
                g = [g[i] + g[i + 1] for i in range(0, len(g), 2)]
            hist1d[pl.ds(blk * 16, 16)] = g[0]

    def merge_pair():
        @plsc.parallel_loop(0, NB // 16, step=1, unroll=4)
        def _mb(blk):
            sl = pl.ds(blk * 16, 16)
            hist1d[sl] = hist1d[sl] + nbh[sl]

    def scan_hist(target):
        def sb(j, carry):
            run, cnt, mx = carry
            blk = NB // 16 - 1 - j
            v = hist1d[pl.ds(blk * 16, 16)]
            cs = plsc.cumsum(lax.rev(v, (0,))) + jnp.broadcast_to(run, (16,))
            ge = cs >= target
            cnt = cnt + jnp.sum(ge.astype(jnp.int32))
            mx = jnp.maximum(mx, jnp.max(jnp.where(ge, 0, cs)))
            return (cs[15], cnt, mx)
        _, cnt, mx = lax.fori_loop(
            0, NB // 16, sb, (jnp.int32(0), jnp.int32(0), jnp.int32(0)))
        return cnt - 1, mx

    zero_hist()
    off1 = jnp.full((16,), -LO1 * S1, jnp.float32)
    hist_pass(a1t, off1)
    reduce_lanes()

    @pl.when(h == 1)
    def _():
        pltpu.sync_copy(hist1d, shist.at[sidx])
    plsc.subcore_barrier()

    @pl.when(h == 0)
    def _():
        pltpu.sync_copy(shist.at[sidx + 1], nbh)
        merge_pair()
        jstar, cabove = scan_hist(jnp.int32(KWIN))
        rem = jnp.int32(KWIN) - cabove
        rbuf[...] = jnp.broadcast_to(rem, (16,))
        lo2 = jnp.float32(LO1) + jstar.astype(jnp.float32) * jnp.float32(W1)
        lbuf[...] = jnp.broadcast_to(lo2, (16,))
        off2 = 1.0 - lo2 * jnp.float32(S2)
        put_params(jnp.broadcast_to(off2, (16,)))
    plsc.subcore_barrier()

    off2v = get_params()
    zero_hist()
    hist_pass(a2t, off2v)
    reduce_lanes()

    @pl.when(h == 1)
    def _():
        pltpu.sync_copy(hist1d, shist.at[sidx])
    plsc.subcore_barrier()

    @pl.when(h == 0)
    def _():
        pltpu.sync_copy(shist.at[sidx + 1], nbh)
        merge_pair()
        blk0 = hist1d[pl.ds(0, 16)]
        hist1d[pl.ds(0, 16)] = jnp.where(lanes == 0, 0, blk0)
        blkt = hist1d[pl.ds(NB - 16, 16)]
        hist1d[pl.ds(NB - 16, 16)] = jnp.where(lanes == 15, 0, blkt)
        rem = rbuf[...][0]
        mstar, _ = scan_hist(rem)
        lo2 = lbuf[...][0]
        tau = lo2 + (mstar - 1).astype(jnp.float32) * jnp.float32(W2)
        put_params(jnp.broadcast_to(tau, (16,)))
    plsc.subcore_barrier()

    tauv = get_params()

    pltpu.async_copy(x_hbm.at[b * C + c0], data0, sin0)

    def outer3(t, _):
        for q in range(2):
            g = t * 2 + q

            @pl.when(g >= 1)
            def _():
                pltpu.make_async_copy(bufs[1 - q],
                                      out_hbm.at[b * C + c0 + g - 1],
                                      souts[1 - q]).wait()

            @pl.when(g + 1 < CH_PER_W)
            def _():
                pltpu.async_copy(x_hbm.at[b * C + c0 + g + 1],
                                 bufs[1 - q], sins[1 - q])
            pltpu.make_async_copy(x_hbm.at[b * C + c0 + g],
                                  bufs[q], sins[q]).wait()
            buf = bufs[q]
            th = tauv * ibft[c0 + g]

            @plsc.parallel_loop(0, 112, step=1, unroll=2)
            def _vb(r):
                for u in range(7):
                    sl = (r, pl.ds(u * 16, 16))
                    v = buf[sl]
                    buf[sl] = jnp.where(v >= th, v, 0.0)
            pltpu.async_copy(buf, out_hbm.at[b * C + c0 + g], souts[q])
        return 0
    lax.fori_loop(0, CH_PER_W // 2, outer3, 0)
    pltpu.make_async_copy(data1, out_hbm.at[b * C + c0 + CH_PER_W - 1],
                          sout1).wait()


@functools.partial(
    pl.kernel,
    out_type=jax.ShapeDtypeStruct((B * C, 112, 112), jnp.float32),
    mesh=plsc.VectorSubcoreMesh(core_axis_name="c", subcore_axis_name="s"),
    scratch_types=[
        pltpu.VMEM((112, 112), jnp.float32),
        pltpu.VMEM((112, 112), jnp.float32),
        pltpu.VMEM((NB * 16,), jnp.int32),
        pltpu.VMEM((NB,), jnp.int32),
        pltpu.VMEM((NB,), jnp.int32),
        pltpu.VMEM((C, 16), jnp.float32),
        pltpu.VMEM((C, 16), jnp.float32),
        pltpu.VMEM((C, 16), jnp.float32),
        pltpu.VMEM((16,), jnp.int32),
        pltpu.VMEM((16,), jnp.int32),
        pltpu.VMEM((16,), jnp.int32),
        pltpu.VMEM((16,), jnp.float32),
        pltpu.VMEM_SHARED((16, NB), jnp.int32),
        pltpu.SemaphoreType.DMA,
        pltpu.SemaphoreType.DMA,
        pltpu.SemaphoreType.DMA,
        pltpu.SemaphoreType.DMA,
    ],
    compiler_params=pltpu.CompilerParams(needs_layout_passes=False),
)
def _kwinners_sc(x_hbm, a1_hbm, a2_hbm, ibf_hbm, out_hbm, *scratch):
    _body(x_hbm, a1_hbm, a2_hbm, ibf_hbm, out_hbm, *scratch)


def kernel(x, dutyCycle):
    x2 = x.reshape(B * C, 112, 112)
    bf = jnp.exp((jnp.float32(TD) - dutyCycle.astype(jnp.float32))).reshape(C, 1)
    a1 = jnp.broadcast_to(bf * jnp.float32(S1), (C, 16))
    a2 = jnp.broadcast_to(bf * jnp.float32(S2), (C, 16))
    ibf = jnp.broadcast_to(1.0 / bf, (C, 16))
    out = _kwinners_sc(x2, a1, a2, ibf)
    return out.reshape(x.shape)
